# Initial kernel scaffold; baseline (speedup 1.0000x reference)
import jax
import jax.numpy as jnp
from jax.experimental import pallas as pl


def kernel(x, Wg, W1, b1, W2, b2):
    raise NotImplementedError("write your pallas kernel here")


# speedup vs baseline: 1.2130x; 1.2130x over previous
import math
import functools

import jax
import jax.numpy as jnp
from jax.experimental import pallas as pl
from jax.experimental.pallas import tpu as pltpu


def _round_up(a, b):
    return (a + b - 1) // b * b


def _router_body(cap, capp, S, x_ref, wg_ref, buf_ref, slot_ref, g_ref):
    x = x_ref[...]
    T, _ = x.shape
    E = wg_ref.shape[1]
    logits = jnp.dot(x, wg_ref[...], preferred_element_type=jnp.float32,
                     precision=jax.lax.Precision.HIGHEST)
    m = jnp.max(logits, axis=-1, keepdims=True)
    ex = jnp.exp(logits - m)
    probs = ex / jnp.sum(ex, axis=-1, keepdims=True)
    g = jnp.max(probs, axis=-1)
    e_idx = jnp.argmax(probs, axis=-1).astype(jnp.int32)

    iota_e = jax.lax.broadcasted_iota(jnp.int32, (T, E), 1)
    oh = (iota_e == e_idx[:, None]).astype(jnp.float32)
    c = oh
    k = 1
    while k < T:
        shifted = jnp.concatenate([jnp.zeros((k, E), jnp.float32), c[:-k]], axis=0)
        c = c + shifted
        k *= 2
    pos = jnp.sum(c * oh, axis=-1).astype(jnp.int32) - 1
    keep = pos < cap
    slot = jnp.where(keep, e_idx * capp + pos, S)

    row_iota = jax.lax.broadcasted_iota(jnp.int32, (S, T), 0)
    P = (row_iota == slot[None, :]).astype(jnp.bfloat16)
    xh = x.astype(jnp.bfloat16)
    xl = (x - xh.astype(jnp.float32)).astype(jnp.bfloat16)
    buf = (jnp.dot(P, xh, preferred_element_type=jnp.float32)
           + jnp.dot(P, xl, preferred_element_type=jnp.float32))
    buf_ref[...] = buf
    slot_ref[...] = slot
    g_ref[...] = g


def _ffn_body(nf, buf_ref, w1_ref, b1_ref, w2_ref, b2_ref, y_ref):
    f = pl.program_id(1)
    a = buf_ref[0]
    h = jnp.dot(a, w1_ref[0], preferred_element_type=jnp.float32) + b1_ref[0]
    h = jax.nn.gelu(h)
    contrib = jnp.dot(h, w2_ref[0], preferred_element_type=jnp.float32)

    @pl.when(f == 0)
    def _():
        y_ref[0] = contrib + b2_ref[0]

    @pl.when(f > 0)
    def _():
        y_ref[0] += contrib


def _combine_body(S, y_ref, slot_ref, g_ref, out_ref):
    T = slot_ref.shape[0]
    slot = slot_ref[...]
    col_iota = jax.lax.broadcasted_iota(jnp.int32, (T, S), 1)
    Pc = (col_iota == slot[:, None]).astype(jnp.bfloat16)
    y = y_ref[...]
    yh = y.astype(jnp.bfloat16)
    yl = (y - yh.astype(jnp.float32)).astype(jnp.bfloat16)
    out = (jnp.dot(Pc, yh, preferred_element_type=jnp.float32)
           + jnp.dot(Pc, yl, preferred_element_type=jnp.float32))
    out_ref[...] = out * g_ref[...][:, None]


def kernel(x, Wg, W1, b1, W2, b2):
    T, D = x.shape
    E = Wg.shape[1]
    F = W1.shape[2]
    cap = int(math.ceil(T * 1 / E * 1.05))
    capp = _round_up(cap, 8)
    S = E * capp
    FB = 1024
    nf = F // FB

    buf, slot, g = pl.pallas_call(
        functools.partial(_router_body, cap, capp, S),
        out_shape=(
            jax.ShapeDtypeStruct((S, D), jnp.float32),
            jax.ShapeDtypeStruct((T,), jnp.int32),
            jax.ShapeDtypeStruct((T,), jnp.float32),
        ),
    )(x, Wg)

    buf3 = buf.reshape(E, capp, D)
    y = pl.pallas_call(
        functools.partial(_ffn_body, nf),
        grid=(E, nf),
        in_specs=[
            pl.BlockSpec((1, capp, D), lambda e, f: (e, 0, 0)),
            pl.BlockSpec((1, D, FB), lambda e, f: (e, 0, f)),
            pl.BlockSpec((1, 1, FB), lambda e, f: (e, 0, f)),
            pl.BlockSpec((1, FB, D), lambda e, f: (e, f, 0)),
            pl.BlockSpec((1, 1, D), lambda e, f: (e, 0, 0)),
        ],
        out_specs=pl.BlockSpec((1, capp, D), lambda e, f: (e, 0, 0)),
        out_shape=jax.ShapeDtypeStruct((E, capp, D), jnp.float32),
    )(buf3, W1, b1.reshape(E, 1, F), W2, b2.reshape(E, 1, D))

    out = pl.pallas_call(
        functools.partial(_combine_body, S),
        out_shape=jax.ShapeDtypeStruct((T, D), jnp.float32),
    )(y.reshape(S, D), slot, g)
    return out


# speedup vs baseline: 1.2589x; 1.0379x over previous
import math
import functools

import jax
import jax.numpy as jnp
from jax.experimental import pallas as pl
from jax.experimental.pallas import tpu as pltpu


def _round_up(a, b):
    return (a + b - 1) // b * b


def _body(cap, capp, S, nf, x_ref, wg_ref, w1_ref, b1_ref, w2_ref, b2_ref,
          out_ref, buf_ref, y_ref, slot_ref, g_ref):
    e = pl.program_id(0)
    f = pl.program_id(1)
    E = wg_ref.shape[1]

    @pl.when((e == 0) & (f == 0))
    def _router():
        x = x_ref[...]
        T = x.shape[0]
        logits = jnp.dot(x, wg_ref[...], preferred_element_type=jnp.float32,
                         precision=jax.lax.Precision.HIGHEST)
        m = jnp.max(logits, axis=-1, keepdims=True)
        ex = jnp.exp(logits - m)
        probs = ex / jnp.sum(ex, axis=-1, keepdims=True)
        g = jnp.max(probs, axis=-1)
        e_idx = jnp.argmax(probs, axis=-1).astype(jnp.int32)

        iota_e = jax.lax.broadcasted_iota(jnp.int32, (T, E), 1)
        oh = (iota_e == e_idx[:, None]).astype(jnp.float32)
        c = oh
        k = 1
        while k < T:
            c = c + jnp.concatenate(
                [jnp.zeros((k, E), jnp.float32), c[:-k]], axis=0)
            k *= 2
        pos = jnp.sum(c * oh, axis=-1).astype(jnp.int32) - 1
        keep = pos < cap
        slot = jnp.where(keep, e_idx * capp + pos, S)

        row_iota = jax.lax.broadcasted_iota(jnp.int32, (S, T), 0)
        P = (row_iota == slot[None, :]).astype(jnp.bfloat16)
        xh = x.astype(jnp.bfloat16)
        xl = (x - xh.astype(jnp.float32)).astype(jnp.bfloat16)
        buf_ref[...] = (jnp.dot(P, xh, preferred_element_type=jnp.float32)
                        + jnp.dot(P, xl, preferred_element_type=jnp.float32))
        slot_ref[...] = slot
        g_ref[...] = g

    a = buf_ref[pl.ds(e * capp, capp), :]
    h = jnp.dot(a, w1_ref[0], preferred_element_type=jnp.float32) + b1_ref[0]
    h = jax.nn.gelu(h)
    contrib = jnp.dot(h, w2_ref[0], preferred_element_type=jnp.float32)

    @pl.when(f == 0)
    def _():
        y_ref[pl.ds(e * capp, capp), :] = contrib + b2_ref[0]

    @pl.when(f > 0)
    def _():
        y_ref[pl.ds(e * capp, capp), :] += contrib

    @pl.when((e == E - 1) & (f == nf - 1))
    def _combine():
        slot = slot_ref[...]
        T = slot.shape[0]
        col_iota = jax.lax.broadcasted_iota(jnp.int32, (T, S), 1)
        Pc = (col_iota == slot[:, None]).astype(jnp.bfloat16)
        y = y_ref[...]
        yh = y.astype(jnp.bfloat16)
        yl = (y - yh.astype(jnp.float32)).astype(jnp.bfloat16)
        out = (jnp.dot(Pc, yh, preferred_element_type=jnp.float32)
               + jnp.dot(Pc, yl, preferred_element_type=jnp.float32))
        out_ref[...] = out * g_ref[...][:, None]


def kernel(x, Wg, W1, b1, W2, b2):
    T, D = x.shape
    E = Wg.shape[1]
    F = W1.shape[2]
    cap = int(math.ceil(T * 1 / E * 1.05))
    capp = _round_up(cap, 8)
    S = E * capp
    FB = 1024
    nf = F // FB

    out = pl.pallas_call(
        functools.partial(_body, cap, capp, S, nf),
        grid=(E, nf),
        in_specs=[
            pl.BlockSpec((T, D), lambda e, f: (0, 0)),
            pl.BlockSpec((D, E), lambda e, f: (0, 0)),
            pl.BlockSpec((1, D, FB), lambda e, f: (e, 0, f)),
            pl.BlockSpec((1, 1, FB), lambda e, f: (e, 0, f)),
            pl.BlockSpec((1, FB, D), lambda e, f: (e, f, 0)),
            pl.BlockSpec((1, 1, D), lambda e, f: (e, 0, 0)),
        ],
        out_specs=pl.BlockSpec((T, D), lambda e, f: (0, 0)),
        out_shape=jax.ShapeDtypeStruct((T, D), jnp.float32),
        scratch_shapes=[
            pltpu.VMEM((S, D), jnp.float32),
            pltpu.VMEM((S, D), jnp.float32),
            pltpu.VMEM((T,), jnp.int32),
            pltpu.VMEM((T,), jnp.float32),
        ],
    )(x, Wg, W1, b1.reshape(E, 1, F), W2, b2.reshape(E, 1, D))
    return out


# speedup vs baseline: 1.2879x; 1.0230x over previous
import math
import functools

import jax
import jax.numpy as jnp
from jax.experimental import pallas as pl
from jax.experimental.pallas import tpu as pltpu


def _round_up(a, b):
    return (a + b - 1) // b * b


def _body(cap, capp, S, nf, x_ref, wg_ref, w1_ref, b1_ref, w2_ref, b2_ref,
          out_ref, buf_ref, y_ref, slot_ref, g_ref):
    e = pl.program_id(0)
    f = pl.program_id(1)
    E = wg_ref.shape[1]

    @pl.when((e == 0) & (f == 0))
    def _router():
        x = x_ref[...]
        T = x.shape[0]
        logits = jnp.dot(x.astype(jnp.bfloat16), wg_ref[...].astype(jnp.bfloat16),
                         preferred_element_type=jnp.float32)
        m = jnp.max(logits, axis=-1, keepdims=True)
        ex = jnp.exp(logits - m)
        probs = ex / jnp.sum(ex, axis=-1, keepdims=True)
        g = jnp.max(probs, axis=-1)
        e_idx = jnp.argmax(probs, axis=-1).astype(jnp.int32)

        iota_e = jax.lax.broadcasted_iota(jnp.int32, (T, E), 1)
        oh = (iota_e == e_idx[:, None]).astype(jnp.float32)
        c = oh
        k = 1
        while k < T:
            c = c + jnp.concatenate(
                [jnp.zeros((k, E), jnp.float32), c[:-k]], axis=0)
            k *= 2
        pos = jnp.sum(c * oh, axis=-1).astype(jnp.int32) - 1
        keep = pos < cap
        slot = jnp.where(keep, e_idx * capp + pos, S)

        row_iota = jax.lax.broadcasted_iota(jnp.int32, (S, T), 0)
        P = (row_iota == slot[None, :]).astype(jnp.bfloat16)
        xh = x.astype(jnp.bfloat16)
        xl = (x - xh.astype(jnp.float32)).astype(jnp.bfloat16)
        buf_ref[...] = (jnp.dot(P, xh, preferred_element_type=jnp.float32)
                        + jnp.dot(P, xl, preferred_element_type=jnp.float32))
        slot_ref[...] = slot
        g_ref[...] = g

    a = buf_ref[pl.ds(e * capp, capp), :]
    h = jnp.dot(a, w1_ref[0], preferred_element_type=jnp.float32) + b1_ref[0]
    h = jax.nn.gelu(h)
    contrib = jnp.dot(h, w2_ref[0], preferred_element_type=jnp.float32)

    @pl.when(f == 0)
    def _():
        y_ref[pl.ds(e * capp, capp), :] = contrib + b2_ref[0]

    @pl.when(f > 0)
    def _():
        y_ref[pl.ds(e * capp, capp), :] += contrib

    @pl.when((e == E - 1) & (f == nf - 1))
    def _combine():
        slot = slot_ref[...]
        T = slot.shape[0]
        col_iota = jax.lax.broadcasted_iota(jnp.int32, (T, S), 1)
        Pc = (col_iota == slot[:, None]).astype(jnp.bfloat16)
        y = y_ref[...]
        yh = y.astype(jnp.bfloat16)
        yl = (y - yh.astype(jnp.float32)).astype(jnp.bfloat16)
        out = (jnp.dot(Pc, yh, preferred_element_type=jnp.float32)
               + jnp.dot(Pc, yl, preferred_element_type=jnp.float32))
        out_ref[...] = out * g_ref[...][:, None]


def kernel(x, Wg, W1, b1, W2, b2):
    T, D = x.shape
    E = Wg.shape[1]
    F = W1.shape[2]
    cap = int(math.ceil(T * 1 / E * 1.05))
    capp = _round_up(cap, 8)
    S = E * capp
    FB = 1024
    nf = F // FB

    out = pl.pallas_call(
        functools.partial(_body, cap, capp, S, nf),
        grid=(E, nf),
        in_specs=[
            pl.BlockSpec((T, D), lambda e, f: (0, 0)),
            pl.BlockSpec((D, E), lambda e, f: (0, 0)),
            pl.BlockSpec((1, D, FB), lambda e, f: (e, 0, f)),
            pl.BlockSpec((1, 1, FB), lambda e, f: (e, 0, f)),
            pl.BlockSpec((1, FB, D), lambda e, f: (e, f, 0)),
            pl.BlockSpec((1, 1, D), lambda e, f: (e, 0, 0)),
        ],
        out_specs=pl.BlockSpec((T, D), lambda e, f: (0, 0)),
        out_shape=jax.ShapeDtypeStruct((T, D), jnp.float32),
        scratch_shapes=[
            pltpu.VMEM((S, D), jnp.float32),
            pltpu.VMEM((S, D), jnp.float32),
            pltpu.VMEM((T,), jnp.int32),
            pltpu.VMEM((T,), jnp.float32),
        ],
    )(x, Wg, W1, b1.reshape(E, 1, F), W2, b2.reshape(E, 1, D))
    return out
